# baseline (speedup 1.0000x reference)
import functools

import jax
import jax.numpy as jnp
from jax import lax
from jax.experimental import pallas as pl
from jax.experimental.pallas import tpu as pltpu
from jax.experimental.pallas import tpu_sc as plsc

H, W = 256, 384
N = H * W
NT = 16
C = N // NT
RPT = C // 128
CC_ITERS = 256


def _prep_body(y_ref, df_ref, dst_ref, fore_ref):
    y = y_ref[:, :]
    dx = df_ref[0]
    dy = df_ref[1]
    ln = jnp.sqrt(dx * dx + dy * dy)
    ux = dx / (ln + 1e-19)
    uy = dy / (ln + 1e-19)
    lc = jnp.clip(ln, 3.0, 6.0)
    rowf = lax.broadcasted_iota(jnp.int32, (H, W), 0).astype(jnp.float32)
    colf = lax.broadcasted_iota(jnp.int32, (H, W), 1).astype(jnp.float32)
    dr = jnp.clip(rowf + ux * lc, 0.0, float(H - 1))
    dc = jnp.clip(colf + uy * lc, 0.0, float(W - 1))
    ri = jnp.round(dr).astype(jnp.int32)
    ci = jnp.round(dc).astype(jnp.int32)
    dst_ref[:, :] = ri * W + ci
    fore_ref[:, :] = (y > 0.5).astype(jnp.float32)


_prep = pl.pallas_call(
    _prep_body,
    out_shape=(
        jax.ShapeDtypeStruct((H, W), jnp.int32),
        jax.ShapeDtypeStruct((H, W), jnp.float32),
    ),
)


def _shv(a, k, fill):
    if k > 0:
        return jnp.concatenate([a[k:], jnp.full((k, W), fill, a.dtype)], 0)
    return jnp.concatenate([jnp.full((-k, W), fill, a.dtype), a[:k]], 0)


def _shh(a, k, fill):
    if k > 0:
        return jnp.concatenate([a[:, k:], jnp.full((H, k), fill, a.dtype)], 1)
    return jnp.concatenate([jnp.full((H, -k), fill, a.dtype), a[:, :k]], 1)


def _dense_body(y_ref, lab_ref):
    x = y_ref[:, :]
    sv = x + _shv(x, 1, 0.0) + _shv(x, -1, 0.0) + _shv(x, 2, 0.0) + _shv(x, -2, 0.0)
    s = sv + _shh(sv, 1, 0.0) + _shh(sv, -1, 0.0) + _shh(sv, 2, 0.0) + _shh(sv, -2, 0.0)
    rowf = lax.broadcasted_iota(jnp.int32, (H, W), 0).astype(jnp.float32)
    colf = lax.broadcasted_iota(jnp.int32, (H, W), 1).astype(jnp.float32)
    cr = jnp.minimum(rowf + 2.0, float(H - 1)) - jnp.maximum(rowf - 2.0, 0.0) + 1.0
    cc = jnp.minimum(colf + 2.0, float(W - 1)) - jnp.maximum(colf - 2.0, 0.0) + 1.0
    q = s / (cr * cc)
    m = (q >= 0.5).astype(jnp.float32)
    e = jnp.minimum(m, _shv(m, 1, 1.0))
    e = jnp.minimum(e, _shv(m, -1, 1.0))
    e = jnp.minimum(e, _shh(m, 1, 1.0))
    e = jnp.minimum(e, _shh(m, -1, 1.0))
    rowi = lax.broadcasted_iota(jnp.int32, (H, W), 0)
    coli = lax.broadcasted_iota(jnp.int32, (H, W), 1)
    lab0 = (rowi * W + coli + 1).astype(jnp.float32) * e

    eb = e > 0

    def step(lab):
        t = jnp.maximum(lab, jnp.maximum(_shv(lab, 1, 0.0), _shv(lab, -1, 0.0)))
        t = jnp.maximum(t, jnp.maximum(_shh(t, 1, 0.0), _shh(t, -1, 0.0)))
        return jnp.where(eb, t, 0.0)

    def wcond(carry):
        k, _, changed = carry
        return jnp.logical_and(k < CC_ITERS // 4, changed)

    def wbody(carry):
        k, lab, _ = carry
        l0 = lab
        for _u in range(4):
            lab = step(lab)
        return (k + 1, lab, jnp.max(lab - l0) > 0.0)

    _, lab_fin, _ = lax.while_loop(
        wcond, wbody, (jnp.int32(0), lab0, jnp.bool_(True)))
    lab_ref[:, :] = lab_fin


_dense = pl.pallas_call(
    _dense_body,
    out_shape=jax.ShapeDtypeStruct((H, W), jnp.float32),
)


@functools.cache
def _sc_mesh():
    return plsc.VectorSubcoreMesh(core_axis_name="c", subcore_axis_name="s",
                                  num_cores=2, num_subcores=16)


def _zero_fill(ref, n):
    def zbody(i, carry):
        ref[pl.ds(i * 16, 16)] = jnp.zeros((16,), jnp.float32)
        return carry

    lax.fori_loop(0, n // 16, zbody, 0)


@functools.cache
def _sc_scatter():
    return functools.partial(
        pl.kernel,
        out_type=jax.ShapeDtypeStruct((N,), jnp.float32),
        mesh=_sc_mesh(),
        scratch_types=[
            pltpu.VMEM((RPT, 128), jnp.int32),
            pltpu.VMEM((C,), jnp.float32),
            pltpu.VMEM((C,), jnp.float32),
            pltpu.VMEM_SHARED((N,), jnp.float32),
            pltpu.VMEM_SHARED((N,), jnp.float32),
        ],
    )(_sc_scatter_body)


def _sc_scatter_body(dst2d, fore, out, idx2, val, y1c, acc_a, acc_b):
    cid = lax.axis_index("c")
    sid = lax.axis_index("s")
    is0 = cid == 0
    base = sid * C

    @pl.when(is0)
    def _stage0():
        pltpu.sync_copy(dst2d.at[pl.ds(sid * RPT, RPT)], idx2)
        _zero_fill(val, C)
        pltpu.sync_copy(val, acc_a.at[pl.ds(base, C)])
        pltpu.sync_copy(val, acc_b.at[pl.ds(base, C)])

    plsc.subcore_barrier()

    @pl.when(is0)
    def _round1():
        pltpu.sync_copy(fore.at[pl.ds(base, C)], val)
        for j in range(RPT):
            pltpu.sync_copy(val.at[pl.ds(j * 128, 128)],
                            acc_a.at[idx2.at[j]], add=True)

    plsc.subcore_barrier()

    @pl.when(is0)
    def _round2():
        pltpu.sync_copy(acc_a.at[pl.ds(base, C)], y1c)
        for j in range(RPT):
            pltpu.sync_copy(y1c.at[pl.ds(j * 128, 128)],
                            acc_b.at[idx2.at[j]], add=True)

    plsc.subcore_barrier()

    @pl.when(is0)
    def _writeout():
        pltpu.sync_copy(acc_b.at[pl.ds(base, C)], out.at[pl.ds(base, C)])


@functools.cache
def _sc_chase():
    return functools.partial(
        pl.kernel,
        out_type=jax.ShapeDtypeStruct((N,), jnp.float32),
        mesh=_sc_mesh(),
        scratch_types=[
            pltpu.VMEM((C,), jnp.int32),
            pltpu.VMEM((C,), jnp.int32),
            pltpu.VMEM((C,), jnp.float32),
            pltpu.VMEM((C,), jnp.float32),
            pltpu.VMEM((C,), jnp.float32),
            pltpu.VMEM((RPT, 128), jnp.int32),
            pltpu.VMEM((C,), jnp.int32),
            pltpu.VMEM((C,), jnp.float32),
            pltpu.VMEM_SHARED((N,), jnp.int32),
            pltpu.VMEM_SHARED((N,), jnp.int32),
            pltpu.VMEM_SHARED((N,), jnp.float32),
            pltpu.VMEM_SHARED((N + 16 + C,), jnp.float32),
        ],
    )(_sc_chase_body)


def _sc_chase_body(dst, inst_a, fore, out,
              myd, tmp, vv, fo, fg, ii2, iif, ob,
              tab_a, tab_b, inst_t, cnts):
    cid = lax.axis_index("c")
    sid = lax.axis_index("s")
    is0 = cid == 0
    base = sid * C

    @pl.when(is0)
    def _stage0():
        pltpu.sync_copy(dst.at[pl.ds(base, C)], myd)
        pltpu.sync_copy(myd, tab_a.at[pl.ds(base, C)])
        pltpu.sync_copy(inst_a.at[pl.ds(base, C)], vv)
        pltpu.sync_copy(vv, inst_t.at[pl.ds(base, C)])
        pltpu.sync_copy(fore.at[pl.ds(base, C)], fo)
        _zero_fill(ob, C)
        pltpu.sync_copy(ob, cnts.at[pl.ds(base, C)])

        @pl.when(sid == 0)
        def _tail():
            pltpu.sync_copy(ob.at[pl.ds(0, 16)], cnts.at[pl.ds(N, 16)])

    plsc.subcore_barrier()

    @pl.when(is0)
    def _double1():
        pltpu.sync_copy(tab_a.at[myd], tmp)
        pltpu.sync_copy(tmp, tab_b.at[pl.ds(base, C)])

    plsc.subcore_barrier()

    @pl.when(is0)
    def _double2():
        pltpu.sync_copy(tab_b.at[tmp], myd)
        pltpu.sync_copy(myd, tab_a.at[pl.ds(base, C)])

    plsc.subcore_barrier()

    @pl.when(is0)
    def _stage3():
        pltpu.sync_copy(tab_a.at[myd], tmp)
        pltpu.sync_copy(inst_t.at[tmp], vv)

        io16 = lax.iota(jnp.int32, 16)

        def ebody(j, carry):
            for k in range(8):
                b = j * 128 + k * 16
                v = vv[pl.ds(b, 16)] * fo[pl.ds(b, 16)]
                vv[pl.ds(b, 16)] = v
                iv = v.astype(jnp.int32)
                iif[pl.ds(b, 16)] = iv
                isfg = v > 0.5
                ii2[j, pl.ds(k * 16, 16)] = jnp.where(isfg, iv, N + 16 + b + io16)
                fg[pl.ds(b, 16)] = jnp.where(isfg, 1.0, 0.0)
            return carry

        lax.fori_loop(0, RPT, ebody, 0)
        for j in range(RPT):
            pltpu.sync_copy(fg.at[pl.ds(j * 128, 128)],
                            cnts.at[ii2.at[j]], add=True)

    plsc.subcore_barrier()

    @pl.when(is0)
    def _stage4():
        pltpu.sync_copy(cnts.at[iif], fo)

        def fbody(j, carry):
            for k in range(8):
                b = j * 128 + k * 16
                ob[pl.ds(b, 16)] = jnp.where(fo[pl.ds(b, 16)] > 256.0,
                                             vv[pl.ds(b, 16)], 0.0)
            return carry

        lax.fori_loop(0, RPT, fbody, 0)
        pltpu.sync_copy(ob, out.at[pl.ds(base, C)])


def kernel(y_hat, df_hat):
    dst_hw, fore_hw = _prep(y_hat, df_hat)
    dst2d = dst_hw.reshape(N // 128, 128)
    dst_flat = dst_hw.reshape(N)
    fore_flat = fore_hw.reshape(N)
    y2 = _sc_scatter()(dst2d, fore_flat)
    inst_a = _dense(y2.reshape(H, W))
    out = _sc_chase()(dst_flat, inst_a.reshape(N), fore_flat)
    return out.reshape(H, W)


# speedup vs baseline: 1.6403x; 1.6403x over previous
import functools

import jax
import jax.numpy as jnp
from jax import lax
from jax.experimental import pallas as pl
from jax.experimental.pallas import tpu as pltpu
from jax.experimental.pallas import tpu_sc as plsc

H, W = 256, 384
N = H * W
NT = 16
C = N // NT
RPT = C // 128
CC_ITERS = 256


def _prep_body(y_ref, df_ref, dst_ref, fore_ref):
    y = y_ref[:, :]
    dx = df_ref[0]
    dy = df_ref[1]
    ln = jnp.sqrt(dx * dx + dy * dy)
    ux = dx / (ln + 1e-19)
    uy = dy / (ln + 1e-19)
    lc = jnp.clip(ln, 3.0, 6.0)
    rowf = lax.broadcasted_iota(jnp.int32, (H, W), 0).astype(jnp.float32)
    colf = lax.broadcasted_iota(jnp.int32, (H, W), 1).astype(jnp.float32)
    dr = jnp.clip(rowf + ux * lc, 0.0, float(H - 1))
    dc = jnp.clip(colf + uy * lc, 0.0, float(W - 1))
    ri = jnp.round(dr).astype(jnp.int32)
    ci = jnp.round(dc).astype(jnp.int32)
    dst_ref[:, :] = ri * W + ci
    fore_ref[:, :] = (y > 0.5).astype(jnp.float32)


_prep = pl.pallas_call(
    _prep_body,
    out_shape=(
        jax.ShapeDtypeStruct((H, W), jnp.int32),
        jax.ShapeDtypeStruct((H, W), jnp.float32),
    ),
)


def _shv(a, k, fill):
    if k > 0:
        return jnp.concatenate([a[k:], jnp.full((k, W), fill, a.dtype)], 0)
    return jnp.concatenate([jnp.full((-k, W), fill, a.dtype), a[:k]], 0)


def _shh(a, k, fill):
    if k > 0:
        return jnp.concatenate([a[:, k:], jnp.full((H, k), fill, a.dtype)], 1)
    return jnp.concatenate([jnp.full((H, -k), fill, a.dtype), a[:, :k]], 1)


def _dense_body(y_ref, lab_ref):
    x = y_ref[:, :]
    sv = x + _shv(x, 1, 0.0) + _shv(x, -1, 0.0) + _shv(x, 2, 0.0) + _shv(x, -2, 0.0)
    s = sv + _shh(sv, 1, 0.0) + _shh(sv, -1, 0.0) + _shh(sv, 2, 0.0) + _shh(sv, -2, 0.0)
    rowf = lax.broadcasted_iota(jnp.int32, (H, W), 0).astype(jnp.float32)
    colf = lax.broadcasted_iota(jnp.int32, (H, W), 1).astype(jnp.float32)
    cr = jnp.minimum(rowf + 2.0, float(H - 1)) - jnp.maximum(rowf - 2.0, 0.0) + 1.0
    cc = jnp.minimum(colf + 2.0, float(W - 1)) - jnp.maximum(colf - 2.0, 0.0) + 1.0
    q = s / (cr * cc)
    m = (q >= 0.5).astype(jnp.float32)
    e = jnp.minimum(m, _shv(m, 1, 1.0))
    e = jnp.minimum(e, _shv(m, -1, 1.0))
    e = jnp.minimum(e, _shh(m, 1, 1.0))
    e = jnp.minimum(e, _shh(m, -1, 1.0))
    rowi = lax.broadcasted_iota(jnp.int32, (H, W), 0)
    coli = lax.broadcasted_iota(jnp.int32, (H, W), 1)
    lab0 = (rowi * W + coli + 1).astype(jnp.float32) * e

    eb = e > 0

    def step(lab):
        t = jnp.maximum(lab, jnp.maximum(_shv(lab, 1, 0.0), _shv(lab, -1, 0.0)))
        t = jnp.maximum(t, jnp.maximum(_shh(t, 1, 0.0), _shh(t, -1, 0.0)))
        return jnp.where(eb, t, 0.0)

    def wcond(carry):
        k, _, changed = carry
        return jnp.logical_and(k < CC_ITERS // 4, changed)

    def wbody(carry):
        k, lab, _ = carry
        l0 = lab
        for _u in range(4):
            lab = step(lab)
        return (k + 1, lab, jnp.max(lab - l0) > 0.0)

    _, lab_fin, _ = lax.while_loop(
        wcond, wbody, (jnp.int32(0), lab0, jnp.bool_(True)))
    lab_ref[:, :] = lab_fin


_dense = pl.pallas_call(
    _dense_body,
    out_shape=jax.ShapeDtypeStruct((H, W), jnp.float32),
)


@functools.cache
def _sc_mesh():
    return plsc.VectorSubcoreMesh(core_axis_name="c", subcore_axis_name="s",
                                  num_cores=2, num_subcores=16)


def _zero_fill(ref, n):
    def zbody(i, carry):
        ref[pl.ds(i * 16, 16)] = jnp.zeros((16,), jnp.float32)
        return carry

    lax.fori_loop(0, n // 16, zbody, 0)


@functools.cache
def _sc_scatter():
    return functools.partial(
        pl.kernel,
        out_type=jax.ShapeDtypeStruct((N,), jnp.float32),
        mesh=_sc_mesh(),
        scratch_types=[
            pltpu.VMEM((RPT, 128), jnp.int32),
            pltpu.VMEM((C,), jnp.float32),
            pltpu.VMEM((C,), jnp.float32),
            pltpu.VMEM_SHARED((N,), jnp.float32),
            pltpu.VMEM_SHARED((N,), jnp.float32),
            pltpu.SemaphoreType.DMA,
        ],
    )(_sc_scatter_body)


def _sc_scatter_body(dst2d, fore, out, idx2, val, y1c, acc_a, acc_b, sem):
    cid = lax.axis_index("c")
    sid = lax.axis_index("s")
    is0 = cid == 0
    base = sid * C

    @pl.when(is0)
    def _stage0():
        pltpu.sync_copy(dst2d.at[pl.ds(sid * RPT, RPT)], idx2)
        _zero_fill(val, C)
        pltpu.sync_copy(val, acc_a.at[pl.ds(base, C)])
        pltpu.sync_copy(val, acc_b.at[pl.ds(base, C)])

    plsc.subcore_barrier()

    @pl.when(is0)
    def _round1():
        pltpu.sync_copy(fore.at[pl.ds(base, C)], val)
        descs = [pltpu.async_copy(val.at[pl.ds(j * 128, 128)],
                                  acc_a.at[idx2.at[j]], sem, add=True)
                 for j in range(RPT)]
        for d in descs:
            d.wait()

    plsc.subcore_barrier()

    @pl.when(is0)
    def _round2():
        pltpu.sync_copy(acc_a.at[pl.ds(base, C)], y1c)
        descs = [pltpu.async_copy(y1c.at[pl.ds(j * 128, 128)],
                                  acc_b.at[idx2.at[j]], sem, add=True)
                 for j in range(RPT)]
        for d in descs:
            d.wait()

    plsc.subcore_barrier()

    @pl.when(is0)
    def _writeout():
        pltpu.sync_copy(acc_b.at[pl.ds(base, C)], out.at[pl.ds(base, C)])


@functools.cache
def _sc_chase():
    return functools.partial(
        pl.kernel,
        out_type=jax.ShapeDtypeStruct((N,), jnp.float32),
        mesh=_sc_mesh(),
        scratch_types=[
            pltpu.VMEM((C,), jnp.int32),
            pltpu.VMEM((C,), jnp.int32),
            pltpu.VMEM((C,), jnp.float32),
            pltpu.VMEM((C,), jnp.float32),
            pltpu.VMEM((C,), jnp.float32),
            pltpu.VMEM((RPT, 128), jnp.int32),
            pltpu.VMEM((C,), jnp.int32),
            pltpu.VMEM((C,), jnp.float32),
            pltpu.VMEM_SHARED((N,), jnp.int32),
            pltpu.VMEM_SHARED((N,), jnp.int32),
            pltpu.VMEM_SHARED((N,), jnp.float32),
            pltpu.VMEM_SHARED((N + 16 + C,), jnp.float32),
            pltpu.SemaphoreType.DMA,
        ],
    )(_sc_chase_body)


def _sc_chase_body(dst, inst_a, fore, out,
                   myd, tmp, vv, fo, fg, ii2, iif, ob,
                   tab_a, tab_b, inst_t, cnts, sem):
    cid = lax.axis_index("c")
    sid = lax.axis_index("s")
    is0 = cid == 0
    base = sid * C

    @pl.when(is0)
    def _stage0():
        pltpu.sync_copy(dst.at[pl.ds(base, C)], myd)
        pltpu.sync_copy(myd, tab_a.at[pl.ds(base, C)])
        pltpu.sync_copy(inst_a.at[pl.ds(base, C)], vv)
        pltpu.sync_copy(vv, inst_t.at[pl.ds(base, C)])
        pltpu.sync_copy(fore.at[pl.ds(base, C)], fo)
        _zero_fill(ob, C)
        pltpu.sync_copy(ob, cnts.at[pl.ds(base, C)])

        @pl.when(sid == 0)
        def _tail():
            pltpu.sync_copy(ob.at[pl.ds(0, 16)], cnts.at[pl.ds(N, 16)])

    plsc.subcore_barrier()

    @pl.when(is0)
    def _double1():
        pltpu.sync_copy(tab_a.at[myd], tmp)
        pltpu.sync_copy(tmp, tab_b.at[pl.ds(base, C)])

    plsc.subcore_barrier()

    @pl.when(is0)
    def _double2():
        pltpu.sync_copy(tab_b.at[tmp], myd)
        pltpu.sync_copy(myd, tab_a.at[pl.ds(base, C)])

    plsc.subcore_barrier()

    @pl.when(is0)
    def _stage3():
        pltpu.sync_copy(tab_a.at[myd], tmp)
        pltpu.sync_copy(inst_t.at[tmp], vv)

        io16 = lax.iota(jnp.int32, 16)

        def ebody(j, carry):
            for k in range(8):
                b = j * 128 + k * 16
                v = vv[pl.ds(b, 16)] * fo[pl.ds(b, 16)]
                vv[pl.ds(b, 16)] = v
                iv = v.astype(jnp.int32)
                isfg = v > 0.5
                siv = jnp.where(isfg, iv, N + 16 + b + io16)
                iif[pl.ds(b, 16)] = siv
                ii2[j, pl.ds(k * 16, 16)] = siv
                fg[pl.ds(b, 16)] = jnp.where(isfg, 1.0, 0.0)
            return carry

        lax.fori_loop(0, RPT, ebody, 0)
        descs = [pltpu.async_copy(fg.at[pl.ds(j * 128, 128)],
                                  cnts.at[ii2.at[j]], sem, add=True)
                 for j in range(RPT)]
        for d in descs:
            d.wait()

    plsc.subcore_barrier()

    @pl.when(is0)
    def _stage4():
        pltpu.sync_copy(cnts.at[iif], fo)

        def fbody(j, carry):
            for k in range(8):
                b = j * 128 + k * 16
                ob[pl.ds(b, 16)] = jnp.where(fo[pl.ds(b, 16)] > 256.0,
                                             vv[pl.ds(b, 16)], 0.0)
            return carry

        lax.fori_loop(0, RPT, fbody, 0)
        pltpu.sync_copy(ob, out.at[pl.ds(base, C)])


def kernel(y_hat, df_hat):
    dst_hw, fore_hw = _prep(y_hat, df_hat)
    dst2d = dst_hw.reshape(N // 128, 128)
    dst_flat = dst_hw.reshape(N)
    fore_flat = fore_hw.reshape(N)
    y2 = _sc_scatter()(dst2d, fore_flat)
    inst_a = _dense(y2.reshape(H, W))
    out = _sc_chase()(dst_flat, inst_a.reshape(N), fore_flat)
    return out.reshape(H, W)


# speedup vs baseline: 1.7831x; 1.0870x over previous
import functools

import jax
import jax.numpy as jnp
from jax import lax
from jax.experimental import pallas as pl
from jax.experimental.pallas import tpu as pltpu
from jax.experimental.pallas import tpu_sc as plsc

H, W = 256, 384
N = H * W
NT = 16
C = N // NT
RPT = C // 128
CC_ITERS = 256


def _prep_body(y_ref, df_ref, dst_ref, fore_ref):
    y = y_ref[:, :]
    dx = df_ref[0]
    dy = df_ref[1]
    ln = jnp.sqrt(dx * dx + dy * dy)
    ux = dx / (ln + 1e-19)
    uy = dy / (ln + 1e-19)
    lc = jnp.clip(ln, 3.0, 6.0)
    rowf = lax.broadcasted_iota(jnp.int32, (H, W), 0).astype(jnp.float32)
    colf = lax.broadcasted_iota(jnp.int32, (H, W), 1).astype(jnp.float32)
    dr = jnp.clip(rowf + ux * lc, 0.0, float(H - 1))
    dc = jnp.clip(colf + uy * lc, 0.0, float(W - 1))
    ri = jnp.round(dr).astype(jnp.int32)
    ci = jnp.round(dc).astype(jnp.int32)
    dst_ref[:, :] = ri * W + ci
    fore_ref[:, :] = (y > 0.5).astype(jnp.float32)


_prep = pl.pallas_call(
    _prep_body,
    out_shape=(
        jax.ShapeDtypeStruct((H, W), jnp.int32),
        jax.ShapeDtypeStruct((H, W), jnp.float32),
    ),
)


def _shv(a, k, fill):
    if k > 0:
        return jnp.concatenate([a[k:], jnp.full((k, W), fill, a.dtype)], 0)
    return jnp.concatenate([jnp.full((-k, W), fill, a.dtype), a[:k]], 0)


def _shh(a, k, fill):
    if k > 0:
        return jnp.concatenate([a[:, k:], jnp.full((H, k), fill, a.dtype)], 1)
    return jnp.concatenate([jnp.full((H, -k), fill, a.dtype), a[:, :k]], 1)


def _dense_body(y_ref, lab_ref):
    x = y_ref[:, :]
    sv = x + _shv(x, 1, 0.0) + _shv(x, -1, 0.0) + _shv(x, 2, 0.0) + _shv(x, -2, 0.0)
    s = sv + _shh(sv, 1, 0.0) + _shh(sv, -1, 0.0) + _shh(sv, 2, 0.0) + _shh(sv, -2, 0.0)
    rowf = lax.broadcasted_iota(jnp.int32, (H, W), 0).astype(jnp.float32)
    colf = lax.broadcasted_iota(jnp.int32, (H, W), 1).astype(jnp.float32)
    cr = jnp.minimum(rowf + 2.0, float(H - 1)) - jnp.maximum(rowf - 2.0, 0.0) + 1.0
    cc = jnp.minimum(colf + 2.0, float(W - 1)) - jnp.maximum(colf - 2.0, 0.0) + 1.0
    q = s / (cr * cc)
    m = (q >= 0.5).astype(jnp.float32)
    e = jnp.minimum(m, _shv(m, 1, 1.0))
    e = jnp.minimum(e, _shv(m, -1, 1.0))
    e = jnp.minimum(e, _shh(m, 1, 1.0))
    e = jnp.minimum(e, _shh(m, -1, 1.0))
    rowi = lax.broadcasted_iota(jnp.int32, (H, W), 0)
    coli = lax.broadcasted_iota(jnp.int32, (H, W), 1)
    lab0 = (rowi * W + coli + 1).astype(jnp.float32) * e

    eb = e > 0

    def step(lab):
        t = jnp.maximum(lab, jnp.maximum(_shv(lab, 1, 0.0), _shv(lab, -1, 0.0)))
        t = jnp.maximum(t, jnp.maximum(_shh(t, 1, 0.0), _shh(t, -1, 0.0)))
        return jnp.where(eb, t, 0.0)

    def wcond(carry):
        k, _, changed = carry
        return jnp.logical_and(k < CC_ITERS // 4, changed)

    def wbody(carry):
        k, lab, _ = carry
        l0 = lab
        for _u in range(4):
            lab = step(lab)
        return (k + 1, lab, jnp.max(lab - l0) > 0.0)

    _, lab_fin, _ = lax.while_loop(
        wcond, wbody, (jnp.int32(0), lab0, jnp.bool_(True)))
    lab_ref[:, :] = lab_fin


_dense = pl.pallas_call(
    _dense_body,
    out_shape=jax.ShapeDtypeStruct((H, W), jnp.float32),
)


@functools.cache
def _sc_mesh():
    return plsc.VectorSubcoreMesh(core_axis_name="c", subcore_axis_name="s",
                                  num_cores=2, num_subcores=16)


def _zero_fill(ref, n):
    def zbody(i, carry):
        ref[pl.ds(i * 16, 16)] = jnp.zeros((16,), jnp.float32)
        return carry

    lax.fori_loop(0, n // 16, zbody, 0)


@functools.cache
def _sc_scatter():
    return functools.partial(
        pl.kernel,
        out_type=jax.ShapeDtypeStruct((N,), jnp.float32),
        mesh=_sc_mesh(),
        scratch_types=[
            pltpu.VMEM((RPT, 128), jnp.int32),
            pltpu.VMEM((C,), jnp.float32),
            pltpu.VMEM((C,), jnp.float32),
            pltpu.VMEM_SHARED((N,), jnp.float32),
            pltpu.VMEM_SHARED((N,), jnp.float32),
            pltpu.SemaphoreType.DMA,
        ],
    )(_sc_scatter_body)


def _sc_scatter_body(dst2d, fore, out, idx2, val, y1c, acc_a, acc_b, sem):
    cid = lax.axis_index("c")
    sid = lax.axis_index("s")
    is0 = cid == 0
    base = sid * C

    @pl.when(is0)
    def _stage0():
        pltpu.sync_copy(dst2d.at[pl.ds(sid * RPT, RPT)], idx2)
        _zero_fill(val, C)
        pltpu.sync_copy(val, acc_a.at[pl.ds(base, C)])
        pltpu.sync_copy(val, acc_b.at[pl.ds(base, C)])

    plsc.subcore_barrier()

    @pl.when(is0)
    def _round1():
        pltpu.sync_copy(fore.at[pl.ds(base, C)], val)
        descs = [pltpu.async_copy(val.at[pl.ds(j * 128, 128)],
                                  acc_a.at[idx2.at[j]], sem, add=True)
                 for j in range(RPT)]
        for d in descs:
            d.wait()

    plsc.subcore_barrier()

    @pl.when(is0)
    def _round2():
        pltpu.sync_copy(acc_a.at[pl.ds(base, C)], y1c)
        descs = [pltpu.async_copy(y1c.at[pl.ds(j * 128, 128)],
                                  acc_b.at[idx2.at[j]], sem, add=True)
                 for j in range(RPT)]
        for d in descs:
            d.wait()

    plsc.subcore_barrier()

    @pl.when(is0)
    def _writeout():
        pltpu.sync_copy(acc_b.at[pl.ds(base, C)], out.at[pl.ds(base, C)])


@functools.cache
def _sc_chain():
    return functools.partial(
        pl.kernel,
        out_type=jax.ShapeDtypeStruct((N,), jnp.int32),
        mesh=_sc_mesh(),
        scratch_types=[
            pltpu.VMEM((C,), jnp.int32),
            pltpu.VMEM((C,), jnp.int32),
            pltpu.VMEM_SHARED((N,), jnp.int32),
            pltpu.VMEM_SHARED((N,), jnp.int32),
        ],
    )(_sc_chain_body)


def _sc_chain_body(dst, out, myd, tmp, tab_a, tab_b):
    cid = lax.axis_index("c")
    sid = lax.axis_index("s")
    is1 = cid == 1
    base = sid * C

    @pl.when(is1)
    def _stage0():
        pltpu.sync_copy(dst.at[pl.ds(base, C)], myd)
        pltpu.sync_copy(myd, tab_a.at[pl.ds(base, C)])

    plsc.subcore_barrier()

    @pl.when(is1)
    def _double1():
        pltpu.sync_copy(tab_a.at[myd], tmp)
        pltpu.sync_copy(tmp, tab_b.at[pl.ds(base, C)])

    plsc.subcore_barrier()

    @pl.when(is1)
    def _double2():
        pltpu.sync_copy(tab_b.at[tmp], myd)
        pltpu.sync_copy(myd, tab_a.at[pl.ds(base, C)])

    plsc.subcore_barrier()

    @pl.when(is1)
    def _double3():
        pltpu.sync_copy(tab_a.at[myd], tmp)
        pltpu.sync_copy(tmp, out.at[pl.ds(base, C)])


@functools.cache
def _sc_final():
    return functools.partial(
        pl.kernel,
        out_type=jax.ShapeDtypeStruct((N,), jnp.float32),
        mesh=_sc_mesh(),
        scratch_types=[
            pltpu.VMEM((C,), jnp.int32),
            pltpu.VMEM((C,), jnp.float32),
            pltpu.VMEM((C,), jnp.float32),
            pltpu.VMEM((C,), jnp.float32),
            pltpu.VMEM((RPT, 128), jnp.int32),
            pltpu.VMEM((C,), jnp.int32),
            pltpu.VMEM((C,), jnp.float32),
            pltpu.VMEM_SHARED((N,), jnp.float32),
            pltpu.VMEM_SHARED((N + 16 + C,), jnp.float32),
            pltpu.SemaphoreType.DMA,
        ],
    )(_sc_final_body)


def _sc_final_body(d8, inst_a, fore, out,
                   myd, vv, fo, fg, ii2, iif, ob,
                   inst_t, cnts, sem):
    cid = lax.axis_index("c")
    sid = lax.axis_index("s")
    is0 = cid == 0
    base = sid * C

    @pl.when(is0)
    def _stage0():
        pltpu.sync_copy(d8.at[pl.ds(base, C)], myd)
        pltpu.sync_copy(inst_a.at[pl.ds(base, C)], vv)
        pltpu.sync_copy(vv, inst_t.at[pl.ds(base, C)])
        pltpu.sync_copy(fore.at[pl.ds(base, C)], fo)
        _zero_fill(ob, C)
        pltpu.sync_copy(ob, cnts.at[pl.ds(base, C)])

        @pl.when(sid == 0)
        def _tail():
            pltpu.sync_copy(ob.at[pl.ds(0, 16)], cnts.at[pl.ds(N, 16)])

    plsc.subcore_barrier()

    @pl.when(is0)
    def _stage1():
        pltpu.sync_copy(inst_t.at[myd], vv)

        io16 = lax.iota(jnp.int32, 16)

        def ebody(j, carry):
            for k in range(8):
                b = j * 128 + k * 16
                v = vv[pl.ds(b, 16)] * fo[pl.ds(b, 16)]
                vv[pl.ds(b, 16)] = v
                iv = v.astype(jnp.int32)
                isfg = v > 0.5
                siv = jnp.where(isfg, iv, N + 16 + b + io16)
                iif[pl.ds(b, 16)] = siv
                ii2[j, pl.ds(k * 16, 16)] = siv
                fg[pl.ds(b, 16)] = jnp.where(isfg, 1.0, 0.0)
            return carry

        lax.fori_loop(0, RPT, ebody, 0)
        descs = [pltpu.async_copy(fg.at[pl.ds(j * 128, 128)],
                                  cnts.at[ii2.at[j]], sem, add=True)
                 for j in range(RPT)]
        for d in descs:
            d.wait()

    plsc.subcore_barrier()

    @pl.when(is0)
    def _stage2():
        pltpu.sync_copy(cnts.at[iif], fo)

        def fbody(j, carry):
            for k in range(8):
                b = j * 128 + k * 16
                ob[pl.ds(b, 16)] = jnp.where(fo[pl.ds(b, 16)] > 256.0,
                                             vv[pl.ds(b, 16)], 0.0)
            return carry

        lax.fori_loop(0, RPT, fbody, 0)
        pltpu.sync_copy(ob, out.at[pl.ds(base, C)])


def kernel(y_hat, df_hat):
    dst_hw, fore_hw = _prep(y_hat, df_hat)
    dst2d = dst_hw.reshape(N // 128, 128)
    dst_flat = dst_hw.reshape(N)
    fore_flat = fore_hw.reshape(N)
    d8 = _sc_chain()(dst_flat)
    y2 = _sc_scatter()(dst2d, fore_flat)
    inst_a = _dense(y2.reshape(H, W))
    out = _sc_final()(d8, inst_a.reshape(N), fore_flat)
    return out.reshape(H, W)
